# Initial kernel scaffold; baseline (speedup 1.0000x reference)
import jax
import jax.numpy as jnp
from jax.experimental import pallas as pl


def kernel(x, edge_index, edge_weight, W1, b1, W2, b2, W3, b3, Wr1, br1, Wr2, br2):
    raise NotImplementedError("write your pallas kernel here")


# speedup vs baseline: 7.3121x; 7.3121x over previous
import functools

import jax
import jax.numpy as jnp
from jax import lax
from jax.experimental import pallas as pl
from jax.experimental.pallas import tpu as pltpu
from jax.experimental.pallas import tpu_sc as plsc

N = 10000
D = 128
E = 320000

NC = 2
NS = 16
NW = NC * NS
L = 16

CHUNK = 128
CPT = -(-E // (NW * CHUNK))
EPT = CPT * CHUNK
EPAD = EPT * NW
NPAD = 10240
RPT = NPAD // NS

_sc_mesh = plsc.VectorSubcoreMesh(
    core_axis_name="c", subcore_axis_name="s", num_cores=NC, num_subcores=NS)


def _zero_rows(rows):
    def _zrow(i, _):
        for q in range(D // L):
            rows[i, pl.ds(q * L, L)] = jnp.zeros((L,), jnp.float32)
        return 0
    lax.fori_loop(0, CHUNK, _zrow, 0)


def _zero_acc(rows, z_sh, s):
    _zero_rows(rows)
    for k in range(RPT // CHUNK):
        pltpu.sync_copy(rows, z_sh.at[pl.ds(s * RPT + k * CHUNK, CHUNK)])
    plsc.subcore_barrier()


def _write_acc(z_sh, z_hbm, c, s):
    plsc.subcore_barrier()
    for k in range(RPT // CHUNK):
        r0 = s * RPT + k * CHUNK
        pltpu.sync_copy(z_sh.at[pl.ds(r0, CHUNK)], z_hbm.at[c, pl.ds(r0, CHUNK)])


def _deg_body(dst_hbm, w_hbm, degp_hbm, dstv, wv, rows, z_sh):
    c = lax.axis_index("c")
    s = lax.axis_index("s")
    wid = c * NS + s
    base = wid * EPT

    _zero_acc(rows, z_sh, s)

    def _chunk(j, _):
        off = base + j * CHUNK
        pltpu.sync_copy(dst_hbm.at[pl.ds(off, CHUNK)], dstv.at[0])
        pltpu.sync_copy(w_hbm.at[pl.ds(off, CHUNK)], wv)

        def _fill(g, _):
            w16 = wv[pl.ds(g * L, L)]
            for j in range(L):
                wb = jnp.full((L,), w16[j])
                for q in range(D // L):
                    rows[g * L + j, pl.ds(q * L, L)] = wb
            return 0
        lax.fori_loop(0, CHUNK // L, _fill, 0)

        pltpu.sync_copy(rows, z_sh.at[dstv.at[0]], add=True)
        return 0
    lax.fori_loop(0, CPT, _chunk, 0)

    _write_acc(z_sh, degp_hbm, c, s)


_deg_kernel = functools.partial(
    pl.kernel,
    out_type=jax.ShapeDtypeStruct((NC, NPAD, D), jnp.float32),
    mesh=_sc_mesh,
    scratch_types=[
        pltpu.VMEM((1, CHUNK), jnp.int32),
        pltpu.VMEM((CHUNK,), jnp.float32),
        pltpu.VMEM((CHUNK, D), jnp.float32),
        pltpu.VMEM_SHARED((NPAD, D), jnp.float32),
    ],
)(_deg_body)


def _edge_body(y_hbm, src_hbm, dst_hbm, w_hbm, z_hbm,
               srcv, dstv, wv, rows, z_sh, sem):
    c = lax.axis_index("c")
    s = lax.axis_index("s")
    wid = c * NS + s
    base = wid * EPT

    _zero_acc(rows, z_sh, s)

    def _chunk(j, _):
        off = base + j * CHUNK
        pltpu.sync_copy(src_hbm.at[pl.ds(off, CHUNK)], srcv)
        pltpu.sync_copy(dst_hbm.at[pl.ds(off, CHUNK)], dstv.at[0])
        pltpu.sync_copy(w_hbm.at[pl.ds(off, CHUNK)], wv)
        pltpu.async_copy(y_hbm.at[srcv], rows, sem).wait()

        def _scale(g, _):
            w16 = wv[pl.ds(g * L, L)]
            for j in range(L):
                i = g * L + j
                wb = jnp.full((L,), w16[j])
                for q in range(D // L):
                    rows[i, pl.ds(q * L, L)] = rows[i, pl.ds(q * L, L)] * wb
            return 0
        lax.fori_loop(0, CHUNK // L, _scale, 0)

        pltpu.sync_copy(rows, z_sh.at[dstv.at[0]], add=True)
        return 0
    lax.fori_loop(0, CPT, _chunk, 0)

    _write_acc(z_sh, z_hbm, c, s)


_edge_kernel = functools.partial(
    pl.kernel,
    out_type=jax.ShapeDtypeStruct((NC, NPAD, D), jnp.float32),
    mesh=_sc_mesh,
    scratch_types=[
        pltpu.VMEM((CHUNK,), jnp.int32),
        pltpu.VMEM((1, CHUNK), jnp.int32),
        pltpu.VMEM((CHUNK,), jnp.float32),
        pltpu.VMEM((CHUNK, D), jnp.float32),
        pltpu.VMEM_SHARED((NPAD, D), jnp.float32),
        pltpu.SemaphoreType.DMA,
    ],
)(_edge_body)


_RB = 2000


def _dinv_body(degp_ref, dinv_ref):
    deg = degp_ref[0, :, 0] + degp_ref[1, :, 0] + 1.0
    dinv_ref[...] = jnp.where(deg > 0, lax.rsqrt(deg), 0.0)[:, None]


def _dinv_kernel(degp):
    blk = 2048
    return pl.pallas_call(
        _dinv_body,
        grid=(NPAD // blk,),
        in_specs=[pl.BlockSpec((NC, blk, D), lambda i: (0, i, 0))],
        out_specs=pl.BlockSpec((blk, 1), lambda i: (i, 0)),
        out_shape=jax.ShapeDtypeStruct((NPAD, 1), jnp.float32),
    )(degp)


def _pre_body(x_ref, w1_ref, dinv_ref, y1_ref):
    xw = jnp.dot(x_ref[...], w1_ref[...], preferred_element_type=jnp.float32)
    y1_ref[...] = xw * dinv_ref[...]


def _pre_kernel(x, w1, dinv):
    return pl.pallas_call(
        _pre_body,
        grid=(N // _RB,),
        in_specs=[
            pl.BlockSpec((_RB, D), lambda i: (i, 0)),
            pl.BlockSpec((D, D), lambda i: (0, 0)),
            pl.BlockSpec((_RB, 1), lambda i: (i, 0)),
        ],
        out_specs=pl.BlockSpec((_RB, D), lambda i: (i, 0)),
        out_shape=jax.ShapeDtypeStruct((N, D), jnp.float32),
    )(x, w1, dinv)


def _mid_body(z_ref, y_ref, dinv_ref, b_ref, w_ref, ynext_ref):
    dv = dinv_ref[...]
    agg = dv * (z_ref[0] + z_ref[1] + y_ref[...]) + b_ref[...]
    h = jnp.maximum(agg, 0.0)
    ynext_ref[...] = jnp.dot(
        h, w_ref[...], preferred_element_type=jnp.float32) * dv


def _mid_kernel(z, y, dinv, b, w):
    return pl.pallas_call(
        _mid_body,
        grid=(N // _RB,),
        in_specs=[
            pl.BlockSpec((NC, _RB, D), lambda i: (0, i, 0)),
            pl.BlockSpec((_RB, D), lambda i: (i, 0)),
            pl.BlockSpec((_RB, 1), lambda i: (i, 0)),
            pl.BlockSpec((1, D), lambda i: (0, 0)),
            pl.BlockSpec((D, D), lambda i: (0, 0)),
        ],
        out_specs=pl.BlockSpec((_RB, D), lambda i: (i, 0)),
        out_shape=jax.ShapeDtypeStruct((N, D), jnp.float32),
    )(z, y, dinv, b, w)


def _head_body(z_ref, y_ref, dinv_ref, b3_ref, wr1_ref, br1_ref, wr2_ref,
               br2_ref, out_ref):
    agg = dinv_ref[...] * (z_ref[0] + z_ref[1] + y_ref[...]) + b3_ref[...]
    r = jnp.maximum(
        jnp.dot(agg, wr1_ref[...], preferred_element_type=jnp.float32)
        + br1_ref[...], 0.0)
    out_ref[...] = jnp.dot(
        r, wr2_ref[...], preferred_element_type=jnp.float32) + br2_ref[...]


def _head_kernel(z, y, dinv, b3, wr1, br1, wr2, br2):
    return pl.pallas_call(
        _head_body,
        grid=(N // _RB,),
        in_specs=[
            pl.BlockSpec((NC, _RB, D), lambda i: (0, i, 0)),
            pl.BlockSpec((_RB, D), lambda i: (i, 0)),
            pl.BlockSpec((_RB, 1), lambda i: (i, 0)),
            pl.BlockSpec((1, D), lambda i: (0, 0)),
            pl.BlockSpec((D, 32), lambda i: (0, 0)),
            pl.BlockSpec((1, 32), lambda i: (0, 0)),
            pl.BlockSpec((32, 1), lambda i: (0, 0)),
            pl.BlockSpec((1, 1), lambda i: (0, 0)),
        ],
        out_specs=pl.BlockSpec((_RB, 1), lambda i: (i, 0)),
        out_shape=jax.ShapeDtypeStruct((N, 1), jnp.float32),
    )(z, y, dinv, b3, wr1, br1, wr2, br2)


def kernel(x, edge_index, edge_weight, W1, b1, W2, b2, W3, b3,
           Wr1, br1, Wr2, br2):
    pad = EPAD - E
    src = jnp.concatenate([edge_index[0], jnp.zeros((pad,), edge_index.dtype)])
    dst = jnp.concatenate([edge_index[1], jnp.zeros((pad,), edge_index.dtype)])
    w = jnp.concatenate([edge_weight, jnp.zeros((pad,), edge_weight.dtype)])

    degp = _deg_kernel(dst, w)
    dinv = _dinv_kernel(degp)
    y = _pre_kernel(x, W1, dinv)
    z = _edge_kernel(y, src, dst, w)
    y = _mid_kernel(z, y, dinv, b1.reshape(1, D), W2)
    z = _edge_kernel(y, src, dst, w)
    y = _mid_kernel(z, y, dinv, b2.reshape(1, D), W3)
    z = _edge_kernel(y, src, dst, w)
    out = _head_kernel(z, y, dinv, b3.reshape(1, D),
                       Wr1, br1.reshape(1, 32), Wr2, br2.reshape(1, 1))
    return out.reshape(N)


# speedup vs baseline: 7.8920x; 1.0793x over previous
import functools

import jax
import jax.numpy as jnp
from jax import lax
from jax.experimental import pallas as pl
from jax.experimental.pallas import tpu as pltpu
from jax.experimental.pallas import tpu_sc as plsc

N = 10000
D = 128
E = 320000

NC = 2
NS = 16
NW = NC * NS
L = 16

CHUNK = 128
CPT = 80
HPT = CPT // 2
EPT = CPT * CHUNK
EPAD = EPT * NW
NPAD = 10240
RPT = NPAD // NS
ZCH = 128

_sc_mesh = plsc.VectorSubcoreMesh(
    core_axis_name="c", subcore_axis_name="s", num_cores=NC, num_subcores=NS)


def _zero_acc(rows, z_sh, s):
    def _zrow(i, _):
        for q in range(D // L):
            rows[i, pl.ds(q * L, L)] = jnp.zeros((L,), jnp.float32)
        return 0
    lax.fori_loop(0, CHUNK, _zrow, 0)
    for k in range(RPT // ZCH):
        pltpu.sync_copy(rows.at[pl.ds(0, ZCH)],
                        z_sh.at[pl.ds(s * RPT + k * ZCH, ZCH)])
    plsc.subcore_barrier()


def _write_acc(z_sh, z_hbm, c, s):
    plsc.subcore_barrier()
    for k in range(RPT // ZCH):
        r0 = s * RPT + k * ZCH
        pltpu.sync_copy(z_sh.at[pl.ds(r0, ZCH)], z_hbm.at[c, pl.ds(r0, ZCH)])


def _fill_chunk(rows, ws, j):
    def _g(g, _):
        w16 = ws[j, pl.ds(g * L, L)]
        for jj in range(L):
            wb = jnp.full((L,), w16[jj])
            for q in range(D // L):
                rows[g * L + jj, pl.ds(q * L, L)] = wb
        return 0
    lax.fori_loop(0, CHUNK // L, _g, 0)


def _scale_chunk(rows, ws, j):
    def _g(g, _):
        w16 = ws[j, pl.ds(g * L, L)]
        for jj in range(L):
            i = g * L + jj
            wb = jnp.full((L,), w16[jj])
            for q in range(D // L):
                rows[i, pl.ds(q * L, L)] = rows[i, pl.ds(q * L, L)] * wb
        return 0
    lax.fori_loop(0, CHUNK // L, _g, 0)


def _deg_body(dst_hbm, w_hbm, degp_hbm, dsts, ws, rows_a, rows_b, z_sh,
              ssem_a, ssem_b):
    c = lax.axis_index("c")
    s = lax.axis_index("s")
    wid = c * NS + s

    _zero_acc(rows_a, z_sh, s)

    for h in range(CPT // HPT):
        pltpu.sync_copy(dst_hbm.at[wid, pl.ds(h * HPT, HPT)], dsts)
        pltpu.sync_copy(w_hbm.at[wid, pl.ds(h * HPT, HPT)], ws)

        def _pair(t, _):
            j0 = 2 * t

            @pl.when(t > 0)
            def _():
                pltpu.make_async_copy(rows_a, z_sh.at[dsts.at[0]], ssem_a).wait()
            _fill_chunk(rows_a, ws, j0)
            pltpu.async_copy(rows_a, z_sh.at[dsts.at[j0]], ssem_a, add=True)

            @pl.when(t > 0)
            def _():
                pltpu.make_async_copy(rows_b, z_sh.at[dsts.at[0]], ssem_b).wait()
            _fill_chunk(rows_b, ws, j0 + 1)
            pltpu.async_copy(rows_b, z_sh.at[dsts.at[j0 + 1]], ssem_b, add=True)
            return 0
        lax.fori_loop(0, HPT // 2, _pair, 0)

        pltpu.make_async_copy(rows_a, z_sh.at[dsts.at[0]], ssem_a).wait()
        pltpu.make_async_copy(rows_b, z_sh.at[dsts.at[0]], ssem_b).wait()

    _write_acc(z_sh, degp_hbm, c, s)


_deg_kernel = functools.partial(
    pl.kernel,
    out_type=jax.ShapeDtypeStruct((NC, NPAD, D), jnp.float32),
    mesh=_sc_mesh,
    scratch_types=[
        pltpu.VMEM((HPT, CHUNK), jnp.int32),
        pltpu.VMEM((HPT, CHUNK), jnp.float32),
        pltpu.VMEM((CHUNK, D), jnp.float32),
        pltpu.VMEM((CHUNK, D), jnp.float32),
        pltpu.VMEM_SHARED((NPAD, D), jnp.float32),
        pltpu.SemaphoreType.DMA,
        pltpu.SemaphoreType.DMA,
    ],
)(_deg_body)


def _edge_body(y_hbm, src_hbm, dst_hbm, w_hbm, z_hbm,
               srcs, dsts, ws, rows_a, rows_b, z_sh,
               gsem_a, gsem_b, ssem_a, ssem_b):
    c = lax.axis_index("c")
    s = lax.axis_index("s")
    wid = c * NS + s

    _zero_acc(rows_a, z_sh, s)

    for h in range(CPT // HPT):
        pltpu.sync_copy(src_hbm.at[wid, pl.ds(h * HPT, HPT)], srcs)
        pltpu.sync_copy(dst_hbm.at[wid, pl.ds(h * HPT, HPT)], dsts)
        pltpu.sync_copy(w_hbm.at[wid, pl.ds(h * HPT, HPT)], ws)

        pltpu.async_copy(y_hbm.at[srcs.at[0]], rows_a, gsem_a)

        def _pair(t, _):
            j0 = 2 * t

            @pl.when(t > 0)
            def _():
                pltpu.make_async_copy(rows_b, z_sh.at[dsts.at[0]], ssem_b).wait()
            pltpu.async_copy(y_hbm.at[srcs.at[j0 + 1]], rows_b, gsem_b)

            pltpu.make_async_copy(y_hbm.at[srcs.at[0]], rows_a, gsem_a).wait()
            _scale_chunk(rows_a, ws, j0)
            pltpu.async_copy(rows_a, z_sh.at[dsts.at[j0]], ssem_a, add=True)

            @pl.when(t + 1 < HPT // 2)
            def _():
                pltpu.make_async_copy(rows_a, z_sh.at[dsts.at[0]], ssem_a).wait()
                pltpu.async_copy(y_hbm.at[srcs.at[j0 + 2]], rows_a, gsem_a)

            pltpu.make_async_copy(y_hbm.at[srcs.at[0]], rows_b, gsem_b).wait()
            _scale_chunk(rows_b, ws, j0 + 1)
            pltpu.async_copy(rows_b, z_sh.at[dsts.at[j0 + 1]], ssem_b, add=True)
            return 0
        lax.fori_loop(0, HPT // 2, _pair, 0)

        pltpu.make_async_copy(rows_a, z_sh.at[dsts.at[0]], ssem_a).wait()
        pltpu.make_async_copy(rows_b, z_sh.at[dsts.at[0]], ssem_b).wait()

    _write_acc(z_sh, z_hbm, c, s)


_edge_kernel = functools.partial(
    pl.kernel,
    out_type=jax.ShapeDtypeStruct((NC, NPAD, D), jnp.float32),
    mesh=_sc_mesh,
    scratch_types=[
        pltpu.VMEM((HPT, CHUNK), jnp.int32),
        pltpu.VMEM((HPT, CHUNK), jnp.int32),
        pltpu.VMEM((HPT, CHUNK), jnp.float32),
        pltpu.VMEM((CHUNK, D), jnp.float32),
        pltpu.VMEM((CHUNK, D), jnp.float32),
        pltpu.VMEM_SHARED((NPAD, D), jnp.float32),
        pltpu.SemaphoreType.DMA,
        pltpu.SemaphoreType.DMA,
        pltpu.SemaphoreType.DMA,
        pltpu.SemaphoreType.DMA,
    ],
)(_edge_body)


_RB = 2000


def _dinv_body(degp_ref, dinv_ref):
    deg = degp_ref[0, :, 0] + degp_ref[1, :, 0] + 1.0
    dinv_ref[...] = jnp.where(deg > 0, lax.rsqrt(deg), 0.0)[:, None]


def _dinv_kernel(degp):
    return pl.pallas_call(
        _dinv_body,
        grid=(N // _RB,),
        in_specs=[pl.BlockSpec((NC, _RB, D), lambda i: (0, i, 0))],
        out_specs=pl.BlockSpec((_RB, 1), lambda i: (i, 0)),
        out_shape=jax.ShapeDtypeStruct((N, 1), jnp.float32),
    )(degp)


def _pre_body(x_ref, w1_ref, dinv_ref, y1_ref):
    xw = jnp.dot(x_ref[...], w1_ref[...], preferred_element_type=jnp.float32)
    y1_ref[...] = xw * dinv_ref[...]


def _pre_kernel(x, w1, dinv):
    return pl.pallas_call(
        _pre_body,
        grid=(N // _RB,),
        in_specs=[
            pl.BlockSpec((_RB, D), lambda i: (i, 0)),
            pl.BlockSpec((D, D), lambda i: (0, 0)),
            pl.BlockSpec((_RB, 1), lambda i: (i, 0)),
        ],
        out_specs=pl.BlockSpec((_RB, D), lambda i: (i, 0)),
        out_shape=jax.ShapeDtypeStruct((N, D), jnp.float32),
    )(x, w1, dinv)


def _mid_body(z_ref, y_ref, dinv_ref, b_ref, w_ref, ynext_ref):
    dv = dinv_ref[...]
    agg = dv * (z_ref[0] + z_ref[1] + y_ref[...]) + b_ref[...]
    h = jnp.maximum(agg, 0.0)
    ynext_ref[...] = jnp.dot(
        h, w_ref[...], preferred_element_type=jnp.float32) * dv


def _mid_kernel(z, y, dinv, b, w):
    return pl.pallas_call(
        _mid_body,
        grid=(N // _RB,),
        in_specs=[
            pl.BlockSpec((NC, _RB, D), lambda i: (0, i, 0)),
            pl.BlockSpec((_RB, D), lambda i: (i, 0)),
            pl.BlockSpec((_RB, 1), lambda i: (i, 0)),
            pl.BlockSpec((1, D), lambda i: (0, 0)),
            pl.BlockSpec((D, D), lambda i: (0, 0)),
        ],
        out_specs=pl.BlockSpec((_RB, D), lambda i: (i, 0)),
        out_shape=jax.ShapeDtypeStruct((N, D), jnp.float32),
    )(z, y, dinv, b, w)


def _head_body(z_ref, y_ref, dinv_ref, b3_ref, wr1_ref, br1_ref, wr2_ref,
               br2_ref, out_ref):
    agg = dinv_ref[...] * (z_ref[0] + z_ref[1] + y_ref[...]) + b3_ref[...]
    r = jnp.maximum(
        jnp.dot(agg, wr1_ref[...], preferred_element_type=jnp.float32)
        + br1_ref[...], 0.0)
    out_ref[...] = jnp.dot(
        r, wr2_ref[...], preferred_element_type=jnp.float32) + br2_ref[...]


def _head_kernel(z, y, dinv, b3, wr1, br1, wr2, br2):
    return pl.pallas_call(
        _head_body,
        grid=(N // _RB,),
        in_specs=[
            pl.BlockSpec((NC, _RB, D), lambda i: (0, i, 0)),
            pl.BlockSpec((_RB, D), lambda i: (i, 0)),
            pl.BlockSpec((_RB, 1), lambda i: (i, 0)),
            pl.BlockSpec((1, D), lambda i: (0, 0)),
            pl.BlockSpec((D, 32), lambda i: (0, 0)),
            pl.BlockSpec((1, 32), lambda i: (0, 0)),
            pl.BlockSpec((32, 1), lambda i: (0, 0)),
            pl.BlockSpec((1, 1), lambda i: (0, 0)),
        ],
        out_specs=pl.BlockSpec((_RB, 1), lambda i: (i, 0)),
        out_shape=jax.ShapeDtypeStruct((N, 1), jnp.float32),
    )(z, y, dinv, b3, wr1, br1, wr2, br2)


def kernel(x, edge_index, edge_weight, W1, b1, W2, b2, W3, b3,
           Wr1, br1, Wr2, br2):
    pad = EPAD - E
    src = jnp.concatenate([edge_index[0], jnp.zeros((pad,), edge_index.dtype)])
    dst = jnp.concatenate([edge_index[1], jnp.zeros((pad,), edge_index.dtype)])
    w = jnp.concatenate([edge_weight, jnp.zeros((pad,), edge_weight.dtype)])
    src = src.reshape(NW, CPT, CHUNK)
    dst = dst.reshape(NW, CPT, CHUNK)
    w = w.reshape(NW, CPT, CHUNK)

    degp = _deg_kernel(dst, w)
    dinv = _dinv_kernel(degp)
    y = _pre_kernel(x, W1, dinv)
    z = _edge_kernel(y, src, dst, w)
    y = _mid_kernel(z, y, dinv, b1.reshape(1, D), W2)
    z = _edge_kernel(y, src, dst, w)
    y = _mid_kernel(z, y, dinv, b2.reshape(1, D), W3)
    z = _edge_kernel(y, src, dst, w)
    out = _head_kernel(z, y, dinv, b3.reshape(1, D),
                       Wr1, br1.reshape(1, 32), Wr2, br2.reshape(1, 1))
    return out.reshape(N)


# speedup vs baseline: 8.0630x; 1.0217x over previous
import functools

import jax
import jax.numpy as jnp
from jax import lax
from jax.experimental import pallas as pl
from jax.experimental.pallas import tpu as pltpu
from jax.experimental.pallas import tpu_sc as plsc

N = 10000
D = 128
E = 320000

NC = 2
NS = 16
NW = NC * NS
L = 16

CHUNK = 128
CPT = 80
HPT = 40
T0 = 120
T1 = 40
TOTCH = NS * (T0 + T1)
EPAD = TOTCH * CHUNK
NPAD = 10240
RPT = NPAD // NS
ZCH = 128

_sc_mesh = plsc.VectorSubcoreMesh(
    core_axis_name="c", subcore_axis_name="s", num_cores=NC, num_subcores=NS)


def _zero_acc(rows, z_sh, s):
    def _zrow(i, _):
        for q in range(D // L):
            rows[i, pl.ds(q * L, L)] = jnp.zeros((L,), jnp.float32)
        return 0
    lax.fori_loop(0, CHUNK, _zrow, 0)
    for k in range(RPT // ZCH):
        pltpu.sync_copy(rows.at[pl.ds(0, ZCH)],
                        z_sh.at[pl.ds(s * RPT + k * ZCH, ZCH)])
    plsc.subcore_barrier()


def _write_acc(z_sh, z_hbm, c, s):
    plsc.subcore_barrier()
    for k in range(RPT // ZCH):
        r0 = s * RPT + k * ZCH
        pltpu.sync_copy(z_sh.at[pl.ds(r0, ZCH)], z_hbm.at[c, pl.ds(r0, ZCH)])


def _fill_chunk(rows, ws, j):
    def _g(g, _):
        w16 = ws[j, pl.ds(g * L, L)]
        for jj in range(L):
            wb = jnp.full((L,), w16[jj])
            for q in range(D // L):
                rows[g * L + jj, pl.ds(q * L, L)] = wb
        return 0
    lax.fori_loop(0, CHUNK // L, _g, 0)


def _scale_chunk(rows, ws, j):
    def _g(g, _):
        w16 = ws[j, pl.ds(g * L, L)]
        for jj in range(L):
            i = g * L + jj
            wb = jnp.full((L,), w16[jj])
            for q in range(D // L):
                rows[i, pl.ds(q * L, L)] = rows[i, pl.ds(q * L, L)] * wb
        return 0
    lax.fori_loop(0, CHUNK // L, _g, 0)


def _deg_body(dst_hbm, w_hbm, degp_hbm, dsts, ws, rows_a, rows_b, z_sh,
              ssem_a, ssem_b):
    c = lax.axis_index("c")
    s = lax.axis_index("s")
    wid = c * NS + s
    cb0 = wid * CPT

    _zero_acc(rows_a, z_sh, s)

    for h in range(CPT // HPT):
        pltpu.sync_copy(dst_hbm.at[pl.ds(cb0 + h * HPT, HPT)], dsts)
        pltpu.sync_copy(w_hbm.at[pl.ds(cb0 + h * HPT, HPT)], ws)

        def _pair(t, _):
            j0 = 2 * t

            @pl.when(t > 0)
            def _():
                pltpu.make_async_copy(rows_a, z_sh.at[dsts.at[0]], ssem_a).wait()
            _fill_chunk(rows_a, ws, j0)
            pltpu.async_copy(rows_a, z_sh.at[dsts.at[j0]], ssem_a, add=True)

            @pl.when(t > 0)
            def _():
                pltpu.make_async_copy(rows_b, z_sh.at[dsts.at[0]], ssem_b).wait()
            _fill_chunk(rows_b, ws, j0 + 1)
            pltpu.async_copy(rows_b, z_sh.at[dsts.at[j0 + 1]], ssem_b, add=True)
            return 0
        lax.fori_loop(0, HPT // 2, _pair, 0)

        pltpu.make_async_copy(rows_a, z_sh.at[dsts.at[0]], ssem_a).wait()
        pltpu.make_async_copy(rows_b, z_sh.at[dsts.at[0]], ssem_b).wait()

    _write_acc(z_sh, degp_hbm, c, s)


_deg_kernel = functools.partial(
    pl.kernel,
    out_type=jax.ShapeDtypeStruct((NC, NPAD, D), jnp.float32),
    mesh=_sc_mesh,
    scratch_types=[
        pltpu.VMEM((HPT, CHUNK), jnp.int32),
        pltpu.VMEM((HPT, CHUNK), jnp.float32),
        pltpu.VMEM((CHUNK, D), jnp.float32),
        pltpu.VMEM((CHUNK, D), jnp.float32),
        pltpu.VMEM_SHARED((NPAD, D), jnp.float32),
        pltpu.SemaphoreType.DMA,
        pltpu.SemaphoreType.DMA,
    ],
)(_deg_body)


def _edge_body(y_hbm, src_hbm, dst_hbm, w_hbm, z_hbm,
               srcs, dsts, ws, rows_a, rows_b, z_sh,
               gsem_a, gsem_b, ssem_a, ssem_b):
    c = lax.axis_index("c")
    s = lax.axis_index("s")
    cb0 = jnp.where(c == 0, s * T0, NS * T0 + s * T1)
    nh = jnp.where(c == 0, T0 // HPT, T1 // HPT)

    _zero_acc(rows_a, z_sh, s)

    def _half(h, _carry):
        cb = cb0 + h * HPT
        pltpu.sync_copy(src_hbm.at[pl.ds(cb, HPT)], srcs)
        pltpu.sync_copy(dst_hbm.at[pl.ds(cb, HPT)], dsts)
        pltpu.sync_copy(w_hbm.at[pl.ds(cb, HPT)], ws)

        pltpu.async_copy(y_hbm.at[srcs.at[0]], rows_a, gsem_a)

        def _pair(t, _):
            j0 = 2 * t

            @pl.when(t > 0)
            def _():
                pltpu.make_async_copy(rows_b, z_sh.at[dsts.at[0]], ssem_b).wait()
            pltpu.async_copy(y_hbm.at[srcs.at[j0 + 1]], rows_b, gsem_b)

            pltpu.make_async_copy(y_hbm.at[srcs.at[0]], rows_a, gsem_a).wait()
            _scale_chunk(rows_a, ws, j0)
            pltpu.async_copy(rows_a, z_sh.at[dsts.at[j0]], ssem_a, add=True)

            @pl.when(t + 1 < HPT // 2)
            def _():
                pltpu.make_async_copy(rows_a, z_sh.at[dsts.at[0]], ssem_a).wait()
                pltpu.async_copy(y_hbm.at[srcs.at[j0 + 2]], rows_a, gsem_a)

            pltpu.make_async_copy(y_hbm.at[srcs.at[0]], rows_b, gsem_b).wait()
            _scale_chunk(rows_b, ws, j0 + 1)
            pltpu.async_copy(rows_b, z_sh.at[dsts.at[j0 + 1]], ssem_b, add=True)
            return 0
        lax.fori_loop(0, HPT // 2, _pair, 0)

        pltpu.make_async_copy(rows_a, z_sh.at[dsts.at[0]], ssem_a).wait()
        pltpu.make_async_copy(rows_b, z_sh.at[dsts.at[0]], ssem_b).wait()
        return 0
    lax.fori_loop(0, nh, _half, 0)

    _write_acc(z_sh, z_hbm, c, s)


_edge_kernel = functools.partial(
    pl.kernel,
    out_type=jax.ShapeDtypeStruct((NC, NPAD, D), jnp.float32),
    mesh=_sc_mesh,
    scratch_types=[
        pltpu.VMEM((HPT, CHUNK), jnp.int32),
        pltpu.VMEM((HPT, CHUNK), jnp.int32),
        pltpu.VMEM((HPT, CHUNK), jnp.float32),
        pltpu.VMEM((CHUNK, D), jnp.float32),
        pltpu.VMEM((CHUNK, D), jnp.float32),
        pltpu.VMEM_SHARED((NPAD, D), jnp.float32),
        pltpu.SemaphoreType.DMA,
        pltpu.SemaphoreType.DMA,
        pltpu.SemaphoreType.DMA,
        pltpu.SemaphoreType.DMA,
    ],
)(_edge_body)


_RB = 2000


def _dinv_body(degp_ref, dinv_ref):
    deg = degp_ref[0, :, 0] + degp_ref[1, :, 0] + 1.0
    dinv_ref[...] = jnp.where(deg > 0, lax.rsqrt(deg), 0.0)[:, None]


def _dinv_kernel(degp):
    return pl.pallas_call(
        _dinv_body,
        grid=(N // _RB,),
        in_specs=[pl.BlockSpec((NC, _RB, D), lambda i: (0, i, 0))],
        out_specs=pl.BlockSpec((_RB, 1), lambda i: (i, 0)),
        out_shape=jax.ShapeDtypeStruct((N, 1), jnp.float32),
    )(degp)


def _pre_body(x_ref, w1_ref, dinv_ref, y1_ref):
    xw = jnp.dot(x_ref[...], w1_ref[...], preferred_element_type=jnp.float32)
    y1_ref[...] = xw * dinv_ref[...]


def _pre_kernel(x, w1, dinv):
    return pl.pallas_call(
        _pre_body,
        grid=(N // _RB,),
        in_specs=[
            pl.BlockSpec((_RB, D), lambda i: (i, 0)),
            pl.BlockSpec((D, D), lambda i: (0, 0)),
            pl.BlockSpec((_RB, 1), lambda i: (i, 0)),
        ],
        out_specs=pl.BlockSpec((_RB, D), lambda i: (i, 0)),
        out_shape=jax.ShapeDtypeStruct((N, D), jnp.float32),
    )(x, w1, dinv)


def _mid_body(z_ref, y_ref, dinv_ref, b_ref, w_ref, ynext_ref):
    dv = dinv_ref[...]
    agg = dv * (z_ref[0] + z_ref[1] + y_ref[...]) + b_ref[...]
    h = jnp.maximum(agg, 0.0)
    ynext_ref[...] = jnp.dot(
        h, w_ref[...], preferred_element_type=jnp.float32) * dv


def _mid_kernel(z, y, dinv, b, w):
    return pl.pallas_call(
        _mid_body,
        grid=(N // _RB,),
        in_specs=[
            pl.BlockSpec((NC, _RB, D), lambda i: (0, i, 0)),
            pl.BlockSpec((_RB, D), lambda i: (i, 0)),
            pl.BlockSpec((_RB, 1), lambda i: (i, 0)),
            pl.BlockSpec((1, D), lambda i: (0, 0)),
            pl.BlockSpec((D, D), lambda i: (0, 0)),
        ],
        out_specs=pl.BlockSpec((_RB, D), lambda i: (i, 0)),
        out_shape=jax.ShapeDtypeStruct((N, D), jnp.float32),
    )(z, y, dinv, b, w)


def _head_body(z_ref, y_ref, dinv_ref, b3_ref, wr1_ref, br1_ref, wr2_ref,
               br2_ref, out_ref):
    agg = dinv_ref[...] * (z_ref[0] + z_ref[1] + y_ref[...]) + b3_ref[...]
    r = jnp.maximum(
        jnp.dot(agg, wr1_ref[...], preferred_element_type=jnp.float32)
        + br1_ref[...], 0.0)
    out_ref[...] = jnp.dot(
        r, wr2_ref[...], preferred_element_type=jnp.float32) + br2_ref[...]


def _head_kernel(z, y, dinv, b3, wr1, br1, wr2, br2):
    return pl.pallas_call(
        _head_body,
        grid=(N // _RB,),
        in_specs=[
            pl.BlockSpec((NC, _RB, D), lambda i: (0, i, 0)),
            pl.BlockSpec((_RB, D), lambda i: (i, 0)),
            pl.BlockSpec((_RB, 1), lambda i: (i, 0)),
            pl.BlockSpec((1, D), lambda i: (0, 0)),
            pl.BlockSpec((D, 32), lambda i: (0, 0)),
            pl.BlockSpec((1, 32), lambda i: (0, 0)),
            pl.BlockSpec((32, 1), lambda i: (0, 0)),
            pl.BlockSpec((1, 1), lambda i: (0, 0)),
        ],
        out_specs=pl.BlockSpec((_RB, 1), lambda i: (i, 0)),
        out_shape=jax.ShapeDtypeStruct((N, 1), jnp.float32),
    )(z, y, dinv, b3, wr1, br1, wr2, br2)


def kernel(x, edge_index, edge_weight, W1, b1, W2, b2, W3, b3,
           Wr1, br1, Wr2, br2):
    pad = EPAD - E
    src = jnp.concatenate([edge_index[0], jnp.zeros((pad,), edge_index.dtype)])
    dst = jnp.concatenate([edge_index[1], jnp.zeros((pad,), edge_index.dtype)])
    w = jnp.concatenate([edge_weight, jnp.zeros((pad,), edge_weight.dtype)])
    src = src.reshape(TOTCH, CHUNK)
    dst = dst.reshape(TOTCH, CHUNK)
    w = w.reshape(TOTCH, CHUNK)

    degp = _deg_kernel(dst, w)
    dinv = _dinv_kernel(degp)
    y = _pre_kernel(x, W1, dinv)
    z = _edge_kernel(y, src, dst, w)
    y = _mid_kernel(z, y, dinv, b1.reshape(1, D), W2)
    z = _edge_kernel(y, src, dst, w)
    y = _mid_kernel(z, y, dinv, b2.reshape(1, D), W3)
    z = _edge_kernel(y, src, dst, w)
    out = _head_kernel(z, y, dinv, b3.reshape(1, D),
                       Wr1, br1.reshape(1, 32), Wr2, br2.reshape(1, 1))
    return out.reshape(N)


# speedup vs baseline: 9.2768x; 1.1505x over previous
import functools

import jax
import jax.numpy as jnp
from jax import lax
from jax.experimental import pallas as pl
from jax.experimental.pallas import tpu as pltpu
from jax.experimental.pallas import tpu_sc as plsc

N = 10000
D = 128
E = 320000

NC = 2
NS = 16
NW = NC * NS
L = 16

CHUNK = 128
CPT = 80
HPT = 16
T0 = 144
T1 = 16
TOTCH = NS * (T0 + T1)
EPAD = TOTCH * CHUNK
NPAD = 10240
RPT = NPAD // NS
ZCH = 128

_sc_mesh = plsc.VectorSubcoreMesh(
    core_axis_name="c", subcore_axis_name="s", num_cores=NC, num_subcores=NS)


def _zero_acc(rows, z_sh, s):
    def _zrow(i, _):
        for q in range(D // L):
            rows[i, pl.ds(q * L, L)] = jnp.zeros((L,), jnp.float32)
        return 0
    lax.fori_loop(0, CHUNK, _zrow, 0)
    for k in range(RPT // ZCH):
        pltpu.sync_copy(rows.at[pl.ds(0, ZCH)],
                        z_sh.at[pl.ds(s * RPT + k * ZCH, ZCH)])
    plsc.subcore_barrier()


def _write_acc(z_sh, z_hbm, c, s):
    plsc.subcore_barrier()
    for k in range(RPT // ZCH):
        r0 = s * RPT + k * ZCH
        pltpu.sync_copy(z_sh.at[pl.ds(r0, ZCH)], z_hbm.at[c, pl.ds(r0, ZCH)])


def _fill_chunk(rows, ws, j):
    def _g(g, _):
        w16 = ws[j, pl.ds(g * L, L)]
        for jj in range(L):
            wb = jnp.full((L,), w16[jj])
            for q in range(D // L):
                rows[g * L + jj, pl.ds(q * L, L)] = wb
        return 0
    lax.fori_loop(0, CHUNK // L, _g, 0)


def _scale_chunk(rows, ws, j):
    def _g(g, _):
        w16 = ws[j, pl.ds(g * L, L)]
        for jj in range(L):
            i = g * L + jj
            wb = jnp.full((L,), w16[jj])
            for q in range(D // L):
                rows[i, pl.ds(q * L, L)] = rows[i, pl.ds(q * L, L)] * wb
        return 0
    lax.fori_loop(0, CHUNK // L, _g, 0)


def _deg_body(dst_hbm, w_hbm, degp_hbm, dsts, ws, rows_a, rows_b, z_sh,
              ssem_a, ssem_b):
    c = lax.axis_index("c")
    s = lax.axis_index("s")
    wid = c * NS + s
    cb0 = wid * CPT

    _zero_acc(rows_a, z_sh, s)

    for h in range(CPT // HPT):
        pltpu.sync_copy(dst_hbm.at[pl.ds(cb0 + h * HPT, HPT)], dsts)
        pltpu.sync_copy(w_hbm.at[pl.ds(cb0 + h * HPT, HPT)], ws)

        def _pair(t, _):
            j0 = 2 * t

            @pl.when(t > 0)
            def _():
                pltpu.make_async_copy(rows_a, z_sh.at[dsts.at[0]], ssem_a).wait()
            _fill_chunk(rows_a, ws, j0)
            pltpu.async_copy(rows_a, z_sh.at[dsts.at[j0]], ssem_a, add=True)

            @pl.when(t > 0)
            def _():
                pltpu.make_async_copy(rows_b, z_sh.at[dsts.at[0]], ssem_b).wait()
            _fill_chunk(rows_b, ws, j0 + 1)
            pltpu.async_copy(rows_b, z_sh.at[dsts.at[j0 + 1]], ssem_b, add=True)
            return 0
        lax.fori_loop(0, HPT // 2, _pair, 0)

        pltpu.make_async_copy(rows_a, z_sh.at[dsts.at[0]], ssem_a).wait()
        pltpu.make_async_copy(rows_b, z_sh.at[dsts.at[0]], ssem_b).wait()

    _write_acc(z_sh, degp_hbm, c, s)


_deg_kernel = functools.partial(
    pl.kernel,
    out_type=jax.ShapeDtypeStruct((NC, NPAD, D), jnp.float32),
    mesh=_sc_mesh,
    scratch_types=[
        pltpu.VMEM((HPT, CHUNK), jnp.int32),
        pltpu.VMEM((HPT, CHUNK), jnp.float32),
        pltpu.VMEM((CHUNK, D), jnp.float32),
        pltpu.VMEM((CHUNK, D), jnp.float32),
        pltpu.VMEM_SHARED((NPAD, D), jnp.float32),
        pltpu.SemaphoreType.DMA,
        pltpu.SemaphoreType.DMA,
    ],
)(_deg_body)


def _edge_body(y_hbm, src_hbm, dst_hbm, w_hbm, z_hbm,
               srcs, dsts, ws, rows_a, rows_b, z_sh,
               gsem_a, gsem_b, ssem_a, ssem_b):
    c = lax.axis_index("c")
    s = lax.axis_index("s")
    cb0 = jnp.where(c == 0, s * T0, NS * T0 + s * T1)
    nh = jnp.where(c == 0, T0 // HPT, T1 // HPT)

    _zero_acc(rows_a, z_sh, s)

    def _half(h, _carry):
        cb = cb0 + h * HPT
        pltpu.sync_copy(src_hbm.at[pl.ds(cb, HPT)], srcs)
        pltpu.sync_copy(dst_hbm.at[pl.ds(cb, HPT)], dsts)
        pltpu.sync_copy(w_hbm.at[pl.ds(cb, HPT)], ws)

        pltpu.async_copy(y_hbm.at[srcs.at[0]], rows_a, gsem_a)

        def _pair(t, _):
            j0 = 2 * t

            @pl.when(t > 0)
            def _():
                pltpu.make_async_copy(rows_b, z_sh.at[dsts.at[0]], ssem_b).wait()
            pltpu.async_copy(y_hbm.at[srcs.at[j0 + 1]], rows_b, gsem_b)

            pltpu.make_async_copy(y_hbm.at[srcs.at[0]], rows_a, gsem_a).wait()
            _scale_chunk(rows_a, ws, j0)
            pltpu.async_copy(rows_a, z_sh.at[dsts.at[j0]], ssem_a, add=True)

            @pl.when(t + 1 < HPT // 2)
            def _():
                pltpu.make_async_copy(rows_a, z_sh.at[dsts.at[0]], ssem_a).wait()
                pltpu.async_copy(y_hbm.at[srcs.at[j0 + 2]], rows_a, gsem_a)

            pltpu.make_async_copy(y_hbm.at[srcs.at[0]], rows_b, gsem_b).wait()
            _scale_chunk(rows_b, ws, j0 + 1)
            pltpu.async_copy(rows_b, z_sh.at[dsts.at[j0 + 1]], ssem_b, add=True)
            return 0
        lax.fori_loop(0, HPT // 2, _pair, 0)

        pltpu.make_async_copy(rows_a, z_sh.at[dsts.at[0]], ssem_a).wait()
        pltpu.make_async_copy(rows_b, z_sh.at[dsts.at[0]], ssem_b).wait()
        return 0
    lax.fori_loop(0, nh, _half, 0)

    _write_acc(z_sh, z_hbm, c, s)


_edge_kernel = functools.partial(
    pl.kernel,
    out_type=jax.ShapeDtypeStruct((NC, NPAD, D), jnp.float32),
    mesh=_sc_mesh,
    scratch_types=[
        pltpu.VMEM((HPT, CHUNK), jnp.int32),
        pltpu.VMEM((HPT, CHUNK), jnp.int32),
        pltpu.VMEM((HPT, CHUNK), jnp.float32),
        pltpu.VMEM((CHUNK, D), jnp.float32),
        pltpu.VMEM((CHUNK, D), jnp.float32),
        pltpu.VMEM_SHARED((NPAD, D), jnp.float32),
        pltpu.SemaphoreType.DMA,
        pltpu.SemaphoreType.DMA,
        pltpu.SemaphoreType.DMA,
        pltpu.SemaphoreType.DMA,
    ],
)(_edge_body)


_RB = 2000


def _dinv_body(degp_ref, dinv_ref):
    deg = degp_ref[0, :, 0] + degp_ref[1, :, 0] + 1.0
    dinv_ref[...] = jnp.where(deg > 0, lax.rsqrt(deg), 0.0)[:, None]


def _dinv_kernel(degp):
    return pl.pallas_call(
        _dinv_body,
        grid=(N // _RB,),
        in_specs=[pl.BlockSpec((NC, _RB, D), lambda i: (0, i, 0))],
        out_specs=pl.BlockSpec((_RB, 1), lambda i: (i, 0)),
        out_shape=jax.ShapeDtypeStruct((N, 1), jnp.float32),
    )(degp)


def _pre_body(x_ref, w1_ref, dinv_ref, y1_ref):
    xw = jnp.dot(x_ref[...], w1_ref[...], preferred_element_type=jnp.float32)
    y1_ref[...] = xw * dinv_ref[...]


def _pre_kernel(x, w1, dinv):
    return pl.pallas_call(
        _pre_body,
        grid=(N // _RB,),
        in_specs=[
            pl.BlockSpec((_RB, D), lambda i: (i, 0)),
            pl.BlockSpec((D, D), lambda i: (0, 0)),
            pl.BlockSpec((_RB, 1), lambda i: (i, 0)),
        ],
        out_specs=pl.BlockSpec((_RB, D), lambda i: (i, 0)),
        out_shape=jax.ShapeDtypeStruct((N, D), jnp.float32),
    )(x, w1, dinv)


def _mid_body(z_ref, y_ref, dinv_ref, b_ref, w_ref, ynext_ref):
    dv = dinv_ref[...]
    agg = dv * (z_ref[0] + z_ref[1] + y_ref[...]) + b_ref[...]
    h = jnp.maximum(agg, 0.0)
    ynext_ref[...] = jnp.dot(
        h, w_ref[...], preferred_element_type=jnp.float32) * dv


def _mid_kernel(z, y, dinv, b, w):
    return pl.pallas_call(
        _mid_body,
        grid=(N // _RB,),
        in_specs=[
            pl.BlockSpec((NC, _RB, D), lambda i: (0, i, 0)),
            pl.BlockSpec((_RB, D), lambda i: (i, 0)),
            pl.BlockSpec((_RB, 1), lambda i: (i, 0)),
            pl.BlockSpec((1, D), lambda i: (0, 0)),
            pl.BlockSpec((D, D), lambda i: (0, 0)),
        ],
        out_specs=pl.BlockSpec((_RB, D), lambda i: (i, 0)),
        out_shape=jax.ShapeDtypeStruct((N, D), jnp.float32),
    )(z, y, dinv, b, w)


def _head_body(z_ref, y_ref, dinv_ref, b3_ref, wr1_ref, br1_ref, wr2_ref,
               br2_ref, out_ref):
    agg = dinv_ref[...] * (z_ref[0] + z_ref[1] + y_ref[...]) + b3_ref[...]
    r = jnp.maximum(
        jnp.dot(agg, wr1_ref[...], preferred_element_type=jnp.float32)
        + br1_ref[...], 0.0)
    out_ref[...] = jnp.dot(
        r, wr2_ref[...], preferred_element_type=jnp.float32) + br2_ref[...]


def _head_kernel(z, y, dinv, b3, wr1, br1, wr2, br2):
    return pl.pallas_call(
        _head_body,
        grid=(N // _RB,),
        in_specs=[
            pl.BlockSpec((NC, _RB, D), lambda i: (0, i, 0)),
            pl.BlockSpec((_RB, D), lambda i: (i, 0)),
            pl.BlockSpec((_RB, 1), lambda i: (i, 0)),
            pl.BlockSpec((1, D), lambda i: (0, 0)),
            pl.BlockSpec((D, 32), lambda i: (0, 0)),
            pl.BlockSpec((1, 32), lambda i: (0, 0)),
            pl.BlockSpec((32, 1), lambda i: (0, 0)),
            pl.BlockSpec((1, 1), lambda i: (0, 0)),
        ],
        out_specs=pl.BlockSpec((_RB, 1), lambda i: (i, 0)),
        out_shape=jax.ShapeDtypeStruct((N, 1), jnp.float32),
    )(z, y, dinv, b3, wr1, br1, wr2, br2)


def kernel(x, edge_index, edge_weight, W1, b1, W2, b2, W3, b3,
           Wr1, br1, Wr2, br2):
    pad = EPAD - E
    src = jnp.concatenate([edge_index[0], jnp.zeros((pad,), edge_index.dtype)])
    dst = jnp.concatenate([edge_index[1], jnp.zeros((pad,), edge_index.dtype)])
    w = jnp.concatenate([edge_weight, jnp.zeros((pad,), edge_weight.dtype)])
    src = src.reshape(TOTCH, CHUNK)
    dst = dst.reshape(TOTCH, CHUNK)
    w = w.reshape(TOTCH, CHUNK)

    degp = _deg_kernel(dst, w)
    dinv = _dinv_kernel(degp)
    y = _pre_kernel(x, W1, dinv)
    z = _edge_kernel(y, src, dst, w)
    y = _mid_kernel(z, y, dinv, b1.reshape(1, D), W2)
    z = _edge_kernel(y, src, dst, w)
    y = _mid_kernel(z, y, dinv, b2.reshape(1, D), W3)
    z = _edge_kernel(y, src, dst, w)
    out = _head_kernel(z, y, dinv, b3.reshape(1, D),
                       Wr1, br1.reshape(1, 32), Wr2, br2.reshape(1, 1))
    return out.reshape(N)
